# Initial kernel scaffold; baseline (speedup 1.0000x reference)
import jax
import jax.numpy as jnp
from jax.experimental import pallas as pl


def kernel(hidden_states, gate_w, w1, w2, w3):
    raise NotImplementedError("write your pallas kernel here")


# speedup vs baseline: 1.6508x; 1.6508x over previous
import functools

import jax
import jax.numpy as jnp
from jax import lax
from jax.experimental import pallas as pl
from jax.experimental.pallas import tpu as pltpu
from jax.experimental.pallas import tpu_sc as plsc

E = 8
K = 2
D = 2048
H = 4096
T = 2 * 2048
A = T * K
BM = 512
G = A // BM + E
P = G * BM
HC = 8
Hc = H // HC

NC, NS = 2, 16
NW = NC * NS
CH = 32


def _router_body(x_ref, gw_ref, ids_ref, wts_ref):
    xb = x_ref[...]
    logits = lax.dot_general(xb, gw_ref[...], (((1,), (1,)), ((), ())),
                             preferred_element_type=jnp.float32)
    m = jnp.max(logits, axis=1, keepdims=True)
    ex = jnp.exp(logits - m)
    p = ex / jnp.sum(ex, axis=1, keepdims=True)
    cols = lax.broadcasted_iota(jnp.int32, p.shape, 1)
    m1 = jnp.max(p, axis=1)
    a1 = jnp.min(jnp.where(p >= m1[:, None], cols, E), axis=1)
    pm = jnp.where(cols == a1[:, None], jnp.float32(-1), p)
    m2 = jnp.max(pm, axis=1)
    a2 = jnp.min(jnp.where(pm >= m2[:, None], cols, E), axis=1)
    s = m1 + m2
    ids_ref[...] = jnp.concatenate([a1[:, None], a2[:, None]], axis=1)
    wts_ref[...] = jnp.concatenate([(m1 / s)[:, None], (m2 / s)[:, None]],
                                   axis=1)


def _router(x, gate_w):
    bt = 512
    return pl.pallas_call(
        _router_body,
        grid=(T // bt,),
        in_specs=[
            pl.BlockSpec((bt, D), lambda i: (i, 0)),
            pl.BlockSpec((E, D), lambda i: (0, 0)),
        ],
        out_specs=[
            pl.BlockSpec((bt, K), lambda i: (i, 0)),
            pl.BlockSpec((bt, K), lambda i: (i, 0)),
        ],
        out_shape=[
            jax.ShapeDtypeStruct((T, K), jnp.int32),
            jax.ShapeDtypeStruct((T, K), jnp.float32),
        ],
    )(x, gate_w)


def _make_row_gather(n_src, n_out):
    rpw = n_out // NW
    iters = rpw // CH
    mesh = plsc.VectorSubcoreMesh(core_axis_name="c", subcore_axis_name="s",
                                  num_cores=NC, num_subcores=NS)

    @functools.partial(
        pl.kernel,
        mesh=mesh,
        out_type=jax.ShapeDtypeStruct((n_out, D), jnp.float32),
        scratch_types=[
            pltpu.VMEM((CH,), jnp.int32),
            pltpu.VMEM((CH, D), jnp.float32),
            pltpu.SemaphoreType.DMA,
        ],
    )
    def gather_k(src_hbm, idx_hbm, out_hbm, idx_v, rows_v, sem):
        wid = lax.axis_index("s") * NC + lax.axis_index("c")
        base0 = wid * rpw
        for i in range(iters):
            base = base0 + i * CH
            pltpu.sync_copy(idx_hbm.at[pl.ds(base, CH)], idx_v)
            pltpu.async_copy(src_hbm.at[idx_v], rows_v, sem).wait()
            pltpu.sync_copy(rows_v, out_hbm.at[pl.ds(base, CH)])

    return gather_k


_gather_x = None
_gather_y = None


def _get_gathers():
    global _gather_x, _gather_y
    if _gather_x is None:
        _gather_x = _make_row_gather(T, P)
        _gather_y = _make_row_gather(P, A)
    return _gather_x, _gather_y


def _ffn_body(be_ref, x_ref, w1_ref, w3_ref, w2_ref, out_ref, acc_ref):
    del be_ref
    hc = pl.program_id(1)
    xb = x_ref[...]
    a1 = lax.dot_general(xb, w1_ref[0], (((1,), (1,)), ((), ())),
                         preferred_element_type=jnp.float32)
    a3 = lax.dot_general(xb, w3_ref[0], (((1,), (1,)), ((), ())),
                         preferred_element_type=jnp.float32)
    h = a1 * jax.nn.sigmoid(a1) * a3
    part = lax.dot_general(h, w2_ref[0], (((1,), (1,)), ((), ())),
                           preferred_element_type=jnp.float32)

    @pl.when(hc == 0)
    def _():
        acc_ref[...] = part

    @pl.when(hc > 0)
    def _():
        acc_ref[...] = acc_ref[...] + part

    @pl.when(hc == HC - 1)
    def _():
        out_ref[...] = acc_ref[...]


def _grouped_ffn(block_expert, x_sorted, w1, w3, w2):
    grid_spec = pltpu.PrefetchScalarGridSpec(
        num_scalar_prefetch=1,
        grid=(G, HC),
        in_specs=[
            pl.BlockSpec((BM, D), lambda g, hc, be: (g, 0)),
            pl.BlockSpec((1, Hc, D), lambda g, hc, be: (be[g], hc, 0)),
            pl.BlockSpec((1, Hc, D), lambda g, hc, be: (be[g], hc, 0)),
            pl.BlockSpec((1, D, Hc), lambda g, hc, be: (be[g], 0, hc)),
        ],
        out_specs=pl.BlockSpec((BM, D), lambda g, hc, be: (g, 0)),
        scratch_shapes=[pltpu.VMEM((BM, D), jnp.float32)],
    )
    return pl.pallas_call(
        _ffn_body,
        grid_spec=grid_spec,
        out_shape=jax.ShapeDtypeStruct((P, D), jnp.float32),
        compiler_params=pltpu.CompilerParams(
            dimension_semantics=("arbitrary", "arbitrary"),
        ),
    )(block_expert, x_sorted, w1, w3, w2)


def _combine_body(y_ref, w_ref, o_ref):
    w0 = w_ref[:, 0:1]
    w1c = w_ref[:, 1:2]
    o_ref[...] = y_ref[:, :D] * w0 + y_ref[:, D:] * w1c


def _combine(y, wts):
    bt = 512
    return pl.pallas_call(
        _combine_body,
        grid=(T // bt,),
        in_specs=[
            pl.BlockSpec((bt, K * D), lambda i: (i, 0)),
            pl.BlockSpec((bt, K), lambda i: (i, 0)),
        ],
        out_specs=pl.BlockSpec((bt, D), lambda i: (i, 0)),
        out_shape=jax.ShapeDtypeStruct((T, D), jnp.float32),
    )(y, wts)


def kernel(hidden_states, gate_w, w1, w2, w3):
    orig_shape = hidden_states.shape
    x = hidden_states.reshape(T, D)

    ids, wts = _router(x, gate_w)

    flat_e = ids.reshape(-1)
    order = jnp.argsort(flat_e)
    e_sorted = flat_e[order]
    counts = jnp.bincount(flat_e, length=E).astype(jnp.int32)
    padded = ((counts + BM - 1) // BM) * BM
    pcs = jnp.cumsum(padded)
    poff = pcs - padded
    start = jnp.cumsum(counts) - counts
    local = jnp.arange(A, dtype=jnp.int32) - start[e_sorted]
    pos = poff[e_sorted] + local
    tok_src = jnp.zeros(P, jnp.int32).at[pos].set(
        (order // K).astype(jnp.int32))
    inv_pos = jnp.zeros(A, jnp.int32).at[order].set(pos)
    block_expert = jnp.clip(
        jnp.searchsorted(pcs, jnp.arange(G, dtype=jnp.int32) * BM,
                         side="right"),
        0, E - 1).astype(jnp.int32)

    gather_x, gather_y = _get_gathers()
    x_sorted = gather_x(x, tok_src)
    contrib = _grouped_ffn(block_expert, x_sorted, w1, w3, w2)
    y = gather_y(contrib, inv_pos).reshape(T, K * D)
    out = _combine(y, wts)
    return out.reshape(orig_shape)
